# baseline (speedup 1.0000x reference)
import functools

import jax
import jax.numpy as jnp
from jax import lax
from jax.experimental import pallas as pl
from jax.experimental.pallas import tpu as pltpu
from jax.experimental.pallas import tpu_sc as plsc

N_NODES = 50000
D_FEAT = 512
NUM_SEGMENTS = 1024
EPSILON = 1e-06

NC = 2
NS = 16
NW = NC * NS
L = 16

R = 400
NBLK = N_NODES // R
GRP = R // L
NGRP = N_NODES // L
CHUNK = ((NGRP + NW - 1) // NW) * L

_CP = pltpu.CompilerParams(needs_layout_passes=False, use_tc_tiling_on_sc=False)

_mesh = plsc.VectorSubcoreMesh(core_axis_name="c", subcore_axis_name="s")


@functools.partial(
    pl.kernel,
    out_type=(
        jax.ShapeDtypeStruct((NUM_SEGMENTS, D_FEAT), jnp.float32),
        jax.ShapeDtypeStruct((NUM_SEGMENTS, D_FEAT), jnp.float32),
        jax.ShapeDtypeStruct((NW * NUM_SEGMENTS,), jnp.float32),
    ),
    mesh=_mesh,
    compiler_params=_CP,
    scratch_types=dict(
        xb=pltpu.VMEM((R, L), jnp.float32),
        idsb=pltpu.VMEM((R,), jnp.int32),
        idsc=pltpu.VMEM((CHUNK,), jnp.int32),
        acc_s=pltpu.VMEM((NUM_SEGMENTS, L), jnp.float32),
        acc_q=pltpu.VMEM((NUM_SEGMENTS, L), jnp.float32),
        acc_c=pltpu.VMEM((NUM_SEGMENTS,), jnp.float32),
    ),
)
def _phase1(x_hbm, ids_hbm, sum_hbm, sq_hbm, cnt_hbm, *, xb, idsb, idsc,
            acc_s, acc_q, acc_c):
    cid = lax.axis_index("c")
    sid = lax.axis_index("s")
    wid = cid * NS + sid
    col0 = wid * L
    iota = lax.iota(jnp.int32, L)
    zeros = jnp.zeros((L,), jnp.float32)
    ones = jnp.ones((L,), jnp.float32)

    def zbody(k, _):
        acc_s[k, pl.ds(0, L)] = zeros
        acc_q[k, pl.ds(0, L)] = zeros
        return 0

    lax.fori_loop(0, NUM_SEGMENTS, zbody, 0)

    def zcbody(k, _):
        acc_c[pl.ds(k * L, L)] = zeros
        return 0

    lax.fori_loop(0, NUM_SEGMENTS // L, zcbody, 0)

    def blk(i, _):
        off = i * R
        pltpu.sync_copy(x_hbm.at[pl.ds(off, R), pl.ds(col0, L)], xb)
        pltpu.sync_copy(ids_hbm.at[pl.ds(off, R)], idsb)

        def grp(g, _):
            idv = idsb[pl.ds(g * L, L)]
            rows = g * L + iota
            for c in range(L):
                cc = jnp.full((L,), c, jnp.int32)
                v = plsc.load_gather(xb, [rows, cc])
                plsc.addupdate_scatter(acc_s, [idv, cc], v)
                plsc.addupdate_scatter(acc_q, [idv, cc], v * v)
            return 0

        lax.fori_loop(0, GRP, grp, 0)
        return 0

    lax.fori_loop(0, NBLK, blk, 0)

    o_w = jnp.minimum(wid * CHUNK, N_NODES - CHUNK)
    lim = wid * CHUNK
    pltpu.sync_copy(ids_hbm.at[pl.ds(o_w, CHUNK)], idsc)

    def cgrp(g, _):
        idv = idsc[pl.ds(g * L, L)]
        ok = (o_w + g * L) >= lim
        m = jnp.full((L,), ok)
        plsc.addupdate_scatter(acc_c, [idv], ones, mask=m)
        return 0

    lax.fori_loop(0, CHUNK // L, cgrp, 0)

    pltpu.sync_copy(acc_s, sum_hbm.at[:, pl.ds(col0, L)])
    pltpu.sync_copy(acc_q, sq_hbm.at[:, pl.ds(col0, L)])
    pltpu.sync_copy(acc_c, cnt_hbm.at[pl.ds(wid * NUM_SEGMENTS, NUM_SEGMENTS)])


def _phase2_body(sum_ref, sq_ref, cnt_ref, bias_ref, scale_ref, c_ref):
    cnt = jnp.sum(cnt_ref[...], axis=0)[:, None]
    n = jnp.maximum(cnt, 1.0)
    bias = bias_ref[0:1, :]
    mean = sum_ref[...] / n
    var = jnp.maximum(sq_ref[...] / n - mean * mean + bias * bias, 0.0)
    scale = lax.rsqrt(var + EPSILON)
    coff = (bias - mean) * scale
    scale_ref[...] = scale
    c_ref[...] = coff


def _phase2(sums, sqs, cnts, bias):
    return pl.pallas_call(
        _phase2_body,
        out_shape=(
            jax.ShapeDtypeStruct((NUM_SEGMENTS, D_FEAT), jnp.float32),
            jax.ShapeDtypeStruct((NUM_SEGMENTS, D_FEAT), jnp.float32),
        ),
    )(sums, sqs, cnts, bias.reshape(1, D_FEAT))


@functools.partial(
    pl.kernel,
    out_type=jax.ShapeDtypeStruct((N_NODES, D_FEAT), jnp.float32),
    mesh=_mesh,
    compiler_params=_CP,
    scratch_types=dict(
        xb=pltpu.VMEM((R, L), jnp.float32),
        ob=pltpu.VMEM((R, L), jnp.float32),
        idsb=pltpu.VMEM((R,), jnp.int32),
        scol=pltpu.VMEM((NUM_SEGMENTS, L), jnp.float32),
        ccol=pltpu.VMEM((NUM_SEGMENTS, L), jnp.float32),
    ),
)
def _phase3(x_hbm, ids_hbm, scale_hbm, coff_hbm, out_hbm, *, xb, ob, idsb,
            scol, ccol):
    cid = lax.axis_index("c")
    sid = lax.axis_index("s")
    wid = cid * NS + sid
    col0 = wid * L
    iota = lax.iota(jnp.int32, L)

    pltpu.sync_copy(scale_hbm.at[:, pl.ds(col0, L)], scol)
    pltpu.sync_copy(coff_hbm.at[:, pl.ds(col0, L)], ccol)

    def blk(i, _):
        off = i * R
        pltpu.sync_copy(x_hbm.at[pl.ds(off, R), pl.ds(col0, L)], xb)
        pltpu.sync_copy(ids_hbm.at[pl.ds(off, R)], idsb)

        def grp(g, _):
            idv = idsb[pl.ds(g * L, L)]
            rows = g * L + iota
            for c in range(L):
                cc = jnp.full((L,), c, jnp.int32)
                v = plsc.load_gather(xb, [rows, cc])
                sv = plsc.load_gather(scol, [idv, cc])
                cv = plsc.load_gather(ccol, [idv, cc])
                plsc.store_scatter(ob, [rows, cc], v * sv + cv)
            return 0

        lax.fori_loop(0, GRP, grp, 0)
        pltpu.sync_copy(ob, out_hbm.at[pl.ds(off, R), pl.ds(col0, L)])
        return 0

    lax.fori_loop(0, NBLK, blk, 0)


@jax.jit
def kernel(inputs, graph_mask, bias):
    seg = graph_mask.astype(jnp.int32)
    sums, sqs, cnts = _phase1(inputs, seg)
    cnts = cnts.reshape(NW, NUM_SEGMENTS)
    scale, coff = _phase2(sums, sqs, cnts, bias)
    return _phase3(inputs, seg, scale, coff)
